# baseline (speedup 1.0000x reference)
import jax
import jax.numpy as jnp
from jax.experimental import pallas as pl
from jax.experimental.pallas import tpu as pltpu

BS, T, BODY_DIM = 32, 256, 263
WIDTH, DEC_WIDTH = 512, 1024
CODE_DIM, NB_CODE, NUM_Q = 512, 1024, 6
DEPTH, DGR = 3, 3

_CONV_PRECISION = None

N_TOK = BS * (T // 2)
TQ = T // 2
SPB = 8
BLK = SPB * TQ
NBLK = N_TOK // BLK


def _conv1d(x, w, b, stride=1, padding=0, dilation=1):
    out = jax.lax.conv_general_dilated(
        x, w, (stride,), [(padding, padding)], rhs_dilation=(dilation,),
        dimension_numbers=('NCH', 'OIH', 'NCH'),
        precision=_CONV_PRECISION)
    return out + b[None, :, None]


def _relu(x):
    return jnp.maximum(x, 0.0)


def _resnet1d(x, p, prefix, reverse=False):
    dils = [DGR ** i for i in range(DEPTH)]
    if reverse:
        dils = dils[::-1]
    for i, d in enumerate(dils):
        h = _relu(x)
        h = _conv1d(h, p[prefix + '_w1_' + str(i)], p[prefix + '_b1_' + str(i)],
                    padding=d, dilation=d)
        h = _relu(h)
        h = _conv1d(h, p[prefix + '_w2_' + str(i)], p[prefix + '_b2_' + str(i)])
        x = x + h
    return x


def _encode(x, p):
    h = _relu(_conv1d(x, p['enc_w0'], p['enc_b0'], padding=1))
    h = _conv1d(h, p['enc_wd'], p['enc_bd'], stride=2, padding=1)
    h = _resnet1d(h, p, 'enc_res')
    h = _conv1d(h, p['enc_wo'], p['enc_bo'], padding=1)
    return h


def _decode(x, p):
    h = _relu(_conv1d(x, p['dec_w0'], p['dec_b0'], padding=1))
    h = _resnet1d(h, p, 'dec_res', reverse=True)
    h = jnp.repeat(h, 2, axis=-1)
    h = _conv1d(h, p['dec_wu'], p['dec_bu'], padding=1)
    h = _relu(_conv1d(h, p['dec_w1'], p['dec_b1'], padding=1))
    h = _conv1d(h, p['dec_wo'], p['dec_bo'], padding=1)
    return h


def _dot(a, b):
    return jax.lax.dot_general(a, b, (((1,), (0,)), ((), ())),
                               preferred_element_type=jnp.float32)


def _split3(x):
    hi = jax.lax.optimization_barrier(x.astype(jnp.bfloat16))
    r1 = x - hi.astype(jnp.float32)
    mid = jax.lax.optimization_barrier(r1.astype(jnp.bfloat16))
    lo = (r1 - mid.astype(jnp.float32)).astype(jnp.bfloat16)
    return hi, mid, lo


def _vq_body(xf_ref, cbt_ref, cb_hi_ref, cb_mid_ref,
             cb_lo_ref, cn_ref, qout_ref, stats_ref, hist_ref, acc_ref):
    pid = pl.program_id(0)

    @pl.when(pid == 0)
    def _init():
        hist_ref[...] = jnp.zeros_like(hist_ref)
        stats_ref[...] = jnp.zeros_like(stats_ref)
        acc_ref[0] = 0.0

    x = xf_ref[...]
    rows = [jnp.transpose(x[s], (1, 0)) for s in range(SPB)]
    HB = BLK // 2
    HS = SPB // 2
    lane_k = jax.lax.broadcasted_iota(jnp.int32, (HB, NB_CODE), 1)
    rs = [jnp.concatenate(rows[0:HS], axis=0),
          jnp.concatenate(rows[HS:SPB], axis=0)]
    qouts = [jnp.zeros_like(rs[0]), jnp.zeros_like(rs[1])]
    commit = jnp.float32(0.0)
    for q in range(NUM_Q):
        mms = [jax.lax.dot_general(
            r, cbt_ref[q], (((1,), (0,)), ((), ())),
            preferred_element_type=jnp.float32) for r in rs]
        ohs = []
        hist_row = None
        for t in range(2):
            d2 = cn_ref[q:q + 1, :] - 2.0 * mms[t]
            m = jnp.min(d2, axis=1, keepdims=True)
            idx = jnp.min(jnp.where(d2 == m, lane_k, NB_CODE), axis=1,
                          keepdims=True)
            onehot = (lane_k == idx).astype(jnp.float32)
            ohs.append(onehot.astype(jnp.bfloat16))
            s = jnp.sum(onehot, axis=0, keepdims=True)
            hist_row = s if hist_row is None else hist_row + s
        hist_ref[q:q + 1, :] += hist_row
        quants = [(_dot(oh, cb_hi_ref[q]) + _dot(oh, cb_mid_ref[q]))
                  + _dot(oh, cb_lo_ref[q]) for oh in ohs]
        for t in range(2):
            r, quant = rs[t], quants[t]
            commit = commit + jnp.sum((r - quant) ** 2)
            qst = r + (quant - r)
            qouts[t] = qouts[t] + qst
            rs[t] = r - qst
    for t in range(2):
        for s in range(HS):
            qout_ref[t * HS + s, :, :] = jnp.transpose(
                qouts[t][s * TQ:(s + 1) * TQ, :], (1, 0))
    acc_ref[0] += commit

    @pl.when(pid == NBLK - 1)
    def _fin():
        probs = hist_ref[...] * (1.0 / N_TOK)
        plog = jnp.log(probs + 1e-10)
        ent = -jnp.sum(probs * plog, axis=1, keepdims=True)
        perp = jnp.sum(jnp.exp(ent)) / NUM_Q
        commit_total = acc_ref[0] / (N_TOK * CODE_DIM)
        row = jax.lax.broadcasted_iota(jnp.int32, (8, 128), 0)
        lane = jax.lax.broadcasted_iota(jnp.int32, (8, 128), 1)
        stats = jnp.where((row == 0) & (lane == 0), commit_total,
                          jnp.where((row == 0) & (lane == 1), perp, 0.0))
        stats_ref[...] = stats


def _residual_vq(x_enc, codebooks):
    cbt = jnp.transpose(codebooks, (0, 2, 1))
    cb_hi, cb_mid, cb_lo = _split3(codebooks)
    cn = jnp.sum(codebooks ** 2, axis=-1)
    qout, stats = pl.pallas_call(
        _vq_body,
        grid=(NBLK,),
        in_specs=[
            pl.BlockSpec((SPB, CODE_DIM, TQ), lambda i: (i, 0, 0)),
            pl.BlockSpec((NUM_Q, CODE_DIM, NB_CODE), lambda i: (0, 0, 0)),
            pl.BlockSpec((NUM_Q, NB_CODE, CODE_DIM), lambda i: (0, 0, 0)),
            pl.BlockSpec((NUM_Q, NB_CODE, CODE_DIM), lambda i: (0, 0, 0)),
            pl.BlockSpec((NUM_Q, NB_CODE, CODE_DIM), lambda i: (0, 0, 0)),
            pl.BlockSpec((NUM_Q, NB_CODE), lambda i: (0, 0)),
        ],
        out_specs=[
            pl.BlockSpec((SPB, CODE_DIM, TQ), lambda i: (i, 0, 0)),
            pl.BlockSpec((8, 128), lambda i: (0, 0)),
        ],
        out_shape=[
            jax.ShapeDtypeStruct((BS, CODE_DIM, TQ), jnp.float32),
            jax.ShapeDtypeStruct((8, 128), jnp.float32),
        ],
        scratch_shapes=[
            pltpu.VMEM((NUM_Q, NB_CODE), jnp.float32),
            pltpu.SMEM((1,), jnp.float32),
        ],
    )(x_enc, cbt, cb_hi, cb_mid, cb_lo, cn)
    return qout, stats[0, 0], stats[0, 1]


def kernel(x_body, codebooks,
           enc_w0, enc_b0, enc_wd, enc_bd,
           enc_res_w1_0, enc_res_b1_0, enc_res_w2_0, enc_res_b2_0,
           enc_res_w1_1, enc_res_b1_1, enc_res_w2_1, enc_res_b2_1,
           enc_res_w1_2, enc_res_b1_2, enc_res_w2_2, enc_res_b2_2,
           enc_wo, enc_bo,
           dec_w0, dec_b0,
           dec_res_w1_0, dec_res_b1_0, dec_res_w2_0, dec_res_b2_0,
           dec_res_w1_1, dec_res_b1_1, dec_res_w2_1, dec_res_b2_1,
           dec_res_w1_2, dec_res_b1_2, dec_res_w2_2, dec_res_b2_2,
           dec_wu, dec_bu, dec_w1, dec_b1, dec_wo, dec_bo):
    inputs = dict(locals())
    p = {k: v for k, v in inputs.items() if k != 'x_body'}
    x_in = jnp.transpose(x_body, (0, 2, 1)).astype(jnp.float32)
    x_enc = _encode(x_in, p)
    xq, commit_loss, perplexity = _residual_vq(x_enc, p['codebooks'])
    x_out = _decode(xq, p)
    return (x_out, commit_loss, perplexity)


# speedup vs baseline: 1.0213x; 1.0213x over previous
import jax
import jax.numpy as jnp
from jax.experimental import pallas as pl
from jax.experimental.pallas import tpu as pltpu

BS, T, BODY_DIM = 32, 256, 263
WIDTH, DEC_WIDTH = 512, 1024
CODE_DIM, NB_CODE, NUM_Q = 512, 1024, 6
DEPTH, DGR = 3, 3

_CONV_PRECISION = None

N_TOK = BS * (T // 2)
BLK = 512
NBLK = N_TOK // BLK


def _conv1d(x, w, b, stride=1, padding=0, dilation=1):
    out = jax.lax.conv_general_dilated(
        x, w, (stride,), [(padding, padding)], rhs_dilation=(dilation,),
        dimension_numbers=('NCH', 'OIH', 'NCH'),
        precision=_CONV_PRECISION)
    return out + b[None, :, None]


def _relu(x):
    return jnp.maximum(x, 0.0)


def _resnet1d(x, p, prefix, reverse=False):
    dils = [DGR ** i for i in range(DEPTH)]
    if reverse:
        dils = dils[::-1]
    for i, d in enumerate(dils):
        h = _relu(x)
        h = _conv1d(h, p[prefix + '_w1_' + str(i)], p[prefix + '_b1_' + str(i)],
                    padding=d, dilation=d)
        h = _relu(h)
        h = _conv1d(h, p[prefix + '_w2_' + str(i)], p[prefix + '_b2_' + str(i)])
        x = x + h
    return x


def _encode(x, p):
    h = _relu(_conv1d(x, p['enc_w0'], p['enc_b0'], padding=1))
    h = _conv1d(h, p['enc_wd'], p['enc_bd'], stride=2, padding=1)
    h = _resnet1d(h, p, 'enc_res')
    h = _conv1d(h, p['enc_wo'], p['enc_bo'], padding=1)
    return h


def _decode(x, p):
    h = _relu(_conv1d(x, p['dec_w0'], p['dec_b0'], padding=1))
    h = _resnet1d(h, p, 'dec_res', reverse=True)
    h = jnp.repeat(h, 2, axis=-1)
    h = _conv1d(h, p['dec_wu'], p['dec_bu'], padding=1)
    h = _relu(_conv1d(h, p['dec_w1'], p['dec_b1'], padding=1))
    h = _conv1d(h, p['dec_wo'], p['dec_bo'], padding=1)
    return h


def _dot(a, b):
    return jax.lax.dot_general(a, b, (((1,), (0,)), ((), ())),
                               preferred_element_type=jnp.float32)


def _split3(x):
    hi = jax.lax.optimization_barrier(x.astype(jnp.bfloat16))
    r1 = x - hi.astype(jnp.float32)
    mid = jax.lax.optimization_barrier(r1.astype(jnp.bfloat16))
    lo = (r1 - mid.astype(jnp.float32)).astype(jnp.bfloat16)
    return hi, mid, lo


def _vq_body(xf_ref, cbf_ref, cb_hi_ref, cb_mid_ref,
             cb_lo_ref, cn_ref, qout_ref, stats_ref, hist_ref, acc_ref):
    pid = pl.program_id(0)

    @pl.when(pid == 0)
    def _init():
        hist_ref[...] = jnp.zeros_like(hist_ref)
        stats_ref[...] = jnp.zeros_like(stats_ref)
        acc_ref[0] = 0.0

    HB = BLK // 2
    lane_k = jax.lax.broadcasted_iota(jnp.int32, (HB, NB_CODE), 1)
    rs = [xf_ref[0:HB, :], xf_ref[HB:BLK, :]]
    qouts = [jnp.zeros_like(rs[0]), jnp.zeros_like(rs[1])]
    commit = jnp.float32(0.0)
    for q in range(NUM_Q):
        mms = [jax.lax.dot_general(
            r, cbf_ref[q], (((1,), (1,)), ((), ())),
            preferred_element_type=jnp.float32) for r in rs]
        ohs = []
        hist_row = None
        for t in range(2):
            d2 = cn_ref[q:q + 1, :] - 2.0 * mms[t]
            m = jnp.min(d2, axis=1, keepdims=True)
            idx = jnp.min(jnp.where(d2 == m, lane_k, NB_CODE), axis=1,
                          keepdims=True)
            onehot = (lane_k == idx).astype(jnp.float32)
            ohs.append(onehot.astype(jnp.bfloat16))
            s = jnp.sum(onehot, axis=0, keepdims=True)
            hist_row = s if hist_row is None else hist_row + s
        hist_ref[q:q + 1, :] += hist_row
        quants = [(_dot(oh, cb_hi_ref[q]) + _dot(oh, cb_mid_ref[q]))
                  + _dot(oh, cb_lo_ref[q]) for oh in ohs]
        for t in range(2):
            r, quant = rs[t], quants[t]
            commit = commit + jnp.sum((r - quant) ** 2)
            qst = r + (quant - r)
            qouts[t] = qouts[t] + qst
            rs[t] = r - qst
    qout_ref[0:HB, :] = qouts[0]
    qout_ref[HB:BLK, :] = qouts[1]
    acc_ref[0] += commit

    @pl.when(pid == NBLK - 1)
    def _fin():
        probs = hist_ref[...] * (1.0 / N_TOK)
        plog = jnp.log(probs + 1e-10)
        ent = -jnp.sum(probs * plog, axis=1, keepdims=True)
        perp = jnp.sum(jnp.exp(ent)) / NUM_Q
        commit_total = acc_ref[0] / (N_TOK * CODE_DIM)
        row = jax.lax.broadcasted_iota(jnp.int32, (8, 128), 0)
        lane = jax.lax.broadcasted_iota(jnp.int32, (8, 128), 1)
        stats = jnp.where((row == 0) & (lane == 0), commit_total,
                          jnp.where((row == 0) & (lane == 1), perp, 0.0))
        stats_ref[...] = stats


def _residual_vq(xf, codebooks):
    cb_hi, cb_mid, cb_lo = _split3(codebooks)
    cn = jnp.sum(codebooks ** 2, axis=-1)
    qout, stats = pl.pallas_call(
        _vq_body,
        grid=(NBLK,),
        in_specs=[
            pl.BlockSpec((BLK, CODE_DIM), lambda i: (i, 0)),
            pl.BlockSpec((NUM_Q, NB_CODE, CODE_DIM), lambda i: (0, 0, 0)),
            pl.BlockSpec((NUM_Q, NB_CODE, CODE_DIM), lambda i: (0, 0, 0)),
            pl.BlockSpec((NUM_Q, NB_CODE, CODE_DIM), lambda i: (0, 0, 0)),
            pl.BlockSpec((NUM_Q, NB_CODE, CODE_DIM), lambda i: (0, 0, 0)),
            pl.BlockSpec((NUM_Q, NB_CODE), lambda i: (0, 0)),
        ],
        out_specs=[
            pl.BlockSpec((BLK, CODE_DIM), lambda i: (i, 0)),
            pl.BlockSpec((8, 128), lambda i: (0, 0)),
        ],
        out_shape=[
            jax.ShapeDtypeStruct((N_TOK, CODE_DIM), jnp.float32),
            jax.ShapeDtypeStruct((8, 128), jnp.float32),
        ],
        scratch_shapes=[
            pltpu.VMEM((NUM_Q, NB_CODE), jnp.float32),
            pltpu.SMEM((1,), jnp.float32),
        ],
    )(xf, codebooks, cb_hi, cb_mid, cb_lo, cn)
    return qout, stats[0, 0], stats[0, 1]


def kernel(x_body, codebooks,
           enc_w0, enc_b0, enc_wd, enc_bd,
           enc_res_w1_0, enc_res_b1_0, enc_res_w2_0, enc_res_b2_0,
           enc_res_w1_1, enc_res_b1_1, enc_res_w2_1, enc_res_b2_1,
           enc_res_w1_2, enc_res_b1_2, enc_res_w2_2, enc_res_b2_2,
           enc_wo, enc_bo,
           dec_w0, dec_b0,
           dec_res_w1_0, dec_res_b1_0, dec_res_w2_0, dec_res_b2_0,
           dec_res_w1_1, dec_res_b1_1, dec_res_w2_1, dec_res_b2_1,
           dec_res_w1_2, dec_res_b1_2, dec_res_w2_2, dec_res_b2_2,
           dec_wu, dec_bu, dec_w1, dec_b1, dec_wo, dec_bo):
    inputs = dict(locals())
    p = {k: v for k, v in inputs.items() if k != 'x_body'}
    x_in = jnp.transpose(x_body, (0, 2, 1)).astype(jnp.float32)
    x_enc = _encode(x_in, p)
    bs, D, Tq = x_enc.shape
    xf = jnp.transpose(x_enc, (0, 2, 1)).reshape(-1, D)
    qout, commit_loss, perplexity = _residual_vq(xf, p['codebooks'])
    xq = jnp.transpose(qout.reshape(bs, Tq, D), (0, 2, 1))
    x_out = _decode(xq, p)
    return (x_out, commit_loss, perplexity)


# speedup vs baseline: 1.0610x; 1.0388x over previous
import jax
import jax.numpy as jnp
from jax.experimental import pallas as pl
from jax.experimental.pallas import tpu as pltpu

BS, T, BODY_DIM = 32, 256, 263
WIDTH, DEC_WIDTH = 512, 1024
CODE_DIM, NB_CODE, NUM_Q = 512, 1024, 6
DEPTH, DGR = 3, 3

_CONV_PRECISION = None

N_TOK = BS * (T // 2)
BLK = 512
NBLK = N_TOK // BLK


def _conv1d(x, w, b, stride=1, padding=0, dilation=1):
    out = jax.lax.conv_general_dilated(
        x, w, (stride,), [(padding, padding)], rhs_dilation=(dilation,),
        dimension_numbers=('NCH', 'OIH', 'NCH'),
        precision=_CONV_PRECISION)
    return out + b[None, :, None]


def _relu(x):
    return jnp.maximum(x, 0.0)


def _resnet1d(x, p, prefix, reverse=False):
    dils = [DGR ** i for i in range(DEPTH)]
    if reverse:
        dils = dils[::-1]
    for i, d in enumerate(dils):
        h = _relu(x)
        h = _conv1d(h, p[prefix + '_w1_' + str(i)], p[prefix + '_b1_' + str(i)],
                    padding=d, dilation=d)
        h = _relu(h)
        h = _conv1d(h, p[prefix + '_w2_' + str(i)], p[prefix + '_b2_' + str(i)])
        x = x + h
    return x


def _encode(x, p):
    h = _relu(_conv1d(x, p['enc_w0'], p['enc_b0'], padding=1))
    h = _conv1d(h, p['enc_wd'], p['enc_bd'], stride=2, padding=1)
    h = _resnet1d(h, p, 'enc_res')
    h = _conv1d(h, p['enc_wo'], p['enc_bo'], padding=1)
    return h


def _decode(x, p):
    h = _relu(_conv1d(x, p['dec_w0'], p['dec_b0'], padding=1))
    h = _resnet1d(h, p, 'dec_res', reverse=True)
    h = jnp.repeat(h, 2, axis=-1)
    h = _conv1d(h, p['dec_wu'], p['dec_bu'], padding=1)
    h = _relu(_conv1d(h, p['dec_w1'], p['dec_b1'], padding=1))
    h = _conv1d(h, p['dec_wo'], p['dec_bo'], padding=1)
    return h


def _dot(a, b):
    return jax.lax.dot_general(a, b, (((1,), (0,)), ((), ())),
                               preferred_element_type=jnp.float32)


def _vq_body(xf_ref, cbf_ref, cn_ref, qout_ref, stats_ref,
             cb_hi_ref, cb_mid_ref, cb_lo_ref, hist_ref, acc_ref):
    pid = pl.program_id(0)

    @pl.when(pid == 0)
    def _init():
        hist_ref[...] = jnp.zeros_like(hist_ref)
        stats_ref[...] = jnp.zeros_like(stats_ref)
        acc_ref[0] = 0.0
        for q in range(NUM_Q):
            x = cbf_ref[q]
            hi = x.astype(jnp.bfloat16)
            r1 = x - hi.astype(jnp.float32)
            mid = r1.astype(jnp.bfloat16)
            lo = (r1 - mid.astype(jnp.float32)).astype(jnp.bfloat16)
            cb_hi_ref[q, :, :] = hi
            cb_mid_ref[q, :, :] = mid
            cb_lo_ref[q, :, :] = lo

    HB = BLK // 2
    lane_k = jax.lax.broadcasted_iota(jnp.int32, (HB, NB_CODE), 1)
    rs = [xf_ref[0:HB, :], xf_ref[HB:BLK, :]]
    qouts = [jnp.zeros_like(rs[0]), jnp.zeros_like(rs[1])]
    commit = jnp.float32(0.0)
    for q in range(NUM_Q):
        mms = [jax.lax.dot_general(
            r, cbf_ref[q], (((1,), (1,)), ((), ())),
            preferred_element_type=jnp.float32) for r in rs]
        ohs = []
        hist_row = None
        for t in range(2):
            d2 = cn_ref[q:q + 1, :] - 2.0 * mms[t]
            m = jnp.min(d2, axis=1, keepdims=True)
            idx = jnp.min(jnp.where(d2 == m, lane_k, NB_CODE), axis=1,
                          keepdims=True)
            onehot = (lane_k == idx).astype(jnp.float32)
            ohs.append(onehot.astype(jnp.bfloat16))
            s = jnp.sum(onehot, axis=0, keepdims=True)
            hist_row = s if hist_row is None else hist_row + s
        hist_ref[q:q + 1, :] += hist_row
        quants = [(_dot(oh, cb_hi_ref[q]) + _dot(oh, cb_mid_ref[q]))
                  + _dot(oh, cb_lo_ref[q]) for oh in ohs]
        for t in range(2):
            r, quant = rs[t], quants[t]
            commit = commit + jnp.sum((r - quant) ** 2)
            qst = r + (quant - r)
            qouts[t] = qouts[t] + qst
            rs[t] = r - qst
    qout_ref[0:HB, :] = qouts[0]
    qout_ref[HB:BLK, :] = qouts[1]
    acc_ref[0] += commit

    @pl.when(pid == NBLK - 1)
    def _fin():
        probs = hist_ref[...] * (1.0 / N_TOK)
        plog = jnp.log(probs + 1e-10)
        ent = -jnp.sum(probs * plog, axis=1, keepdims=True)
        perp = jnp.sum(jnp.exp(ent)) / NUM_Q
        commit_total = acc_ref[0] / (N_TOK * CODE_DIM)
        row = jax.lax.broadcasted_iota(jnp.int32, (8, 128), 0)
        lane = jax.lax.broadcasted_iota(jnp.int32, (8, 128), 1)
        stats = jnp.where((row == 0) & (lane == 0), commit_total,
                          jnp.where((row == 0) & (lane == 1), perp, 0.0))
        stats_ref[...] = stats


def _residual_vq(xf, codebooks):
    cn = jnp.sum(codebooks ** 2, axis=-1)
    qout, stats = pl.pallas_call(
        _vq_body,
        grid=(NBLK,),
        in_specs=[
            pl.BlockSpec((BLK, CODE_DIM), lambda i: (i, 0)),
            pl.BlockSpec((NUM_Q, NB_CODE, CODE_DIM), lambda i: (0, 0, 0)),
            pl.BlockSpec((NUM_Q, NB_CODE), lambda i: (0, 0)),
        ],
        out_specs=[
            pl.BlockSpec((BLK, CODE_DIM), lambda i: (i, 0)),
            pl.BlockSpec((8, 128), lambda i: (0, 0)),
        ],
        out_shape=[
            jax.ShapeDtypeStruct((N_TOK, CODE_DIM), jnp.float32),
            jax.ShapeDtypeStruct((8, 128), jnp.float32),
        ],
        scratch_shapes=[
            pltpu.VMEM((NUM_Q, NB_CODE, CODE_DIM), jnp.bfloat16),
            pltpu.VMEM((NUM_Q, NB_CODE, CODE_DIM), jnp.bfloat16),
            pltpu.VMEM((NUM_Q, NB_CODE, CODE_DIM), jnp.bfloat16),
            pltpu.VMEM((NUM_Q, NB_CODE), jnp.float32),
            pltpu.SMEM((1,), jnp.float32),
        ],
    )(xf, codebooks, cn)
    return qout, stats[0, 0], stats[0, 1]


def kernel(x_body, codebooks,
           enc_w0, enc_b0, enc_wd, enc_bd,
           enc_res_w1_0, enc_res_b1_0, enc_res_w2_0, enc_res_b2_0,
           enc_res_w1_1, enc_res_b1_1, enc_res_w2_1, enc_res_b2_1,
           enc_res_w1_2, enc_res_b1_2, enc_res_w2_2, enc_res_b2_2,
           enc_wo, enc_bo,
           dec_w0, dec_b0,
           dec_res_w1_0, dec_res_b1_0, dec_res_w2_0, dec_res_b2_0,
           dec_res_w1_1, dec_res_b1_1, dec_res_w2_1, dec_res_b2_1,
           dec_res_w1_2, dec_res_b1_2, dec_res_w2_2, dec_res_b2_2,
           dec_wu, dec_bu, dec_w1, dec_b1, dec_wo, dec_bo):
    inputs = dict(locals())
    p = {k: v for k, v in inputs.items() if k != 'x_body'}
    x_in = jnp.transpose(x_body, (0, 2, 1)).astype(jnp.float32)
    x_enc = _encode(x_in, p)
    bs, D, Tq = x_enc.shape
    xf = jnp.transpose(x_enc, (0, 2, 1)).reshape(-1, D)
    qout, commit_loss, perplexity = _residual_vq(xf, p['codebooks'])
    xq = jnp.transpose(qout.reshape(bs, Tq, D), (0, 2, 1))
    x_out = _decode(xq, p)
    return (x_out, commit_loss, perplexity)
